# Initial kernel scaffold; baseline (speedup 1.0000x reference)
import jax
import jax.numpy as jnp
from jax.experimental import pallas as pl


def kernel(embedding_features, reference_embeddings, auxiliary_features):
    raise NotImplementedError("write your pallas kernel here")


# speedup vs baseline: 1.7844x; 1.7844x over previous
import functools

import jax
import jax.numpy as jnp
from jax import lax
from jax.experimental import pallas as pl
from jax.experimental.pallas import tpu as pltpu
from jax.experimental.pallas import tpu_sc as plsc

Q = 1024
D = 16
D_AUX = 64
KNN = 5

INF_F = float("inf")
BIG_I = 2**30


def _topk_kernel(a_ref, bT_ref, vals_ref, idx_ref, *, nblk, blk, k_total):
    j = pl.program_id(0)

    a = a_ref[...]
    bT = bT_ref[...]
    b2 = jnp.sum(bT * bT, axis=0, keepdims=True)
    ab = lax.dot_general(a, bT, (((1,), (0,)), ((), ())),
                         preferred_element_type=jnp.float32)
    d2 = b2 - 2.0 * ab

    colg = jax.lax.broadcasted_iota(jnp.int32, (1, blk), 1) + j * blk
    d2m = jnp.where(colg < k_total, d2, INF_F)

    blk_v, blk_i = [], []
    for _ in range(KNN):
        m = jnp.min(d2m, axis=1, keepdims=True)
        sel = jnp.where(d2m == m, colg, BIG_I)
        am = jnp.min(sel, axis=1, keepdims=True)
        blk_v.append(m)
        blk_i.append(am)
        d2m = jnp.where(colg == am, INF_F, d2m)

    first = j == 0
    cur_v = jnp.where(first, INF_F, vals_ref[...])
    cur_i = jnp.where(first, 0, idx_ref[...])

    comb_v = jnp.concatenate([cur_v] + blk_v, axis=1)
    comb_i = jnp.concatenate([cur_i] + blk_i, axis=1)
    pos = jax.lax.broadcasted_iota(jnp.int32, (1, 13), 1)

    new_v, new_i = [], []
    for _ in range(KNN):
        m = jnp.min(comb_v, axis=1, keepdims=True)
        p = jnp.min(jnp.where(comb_v == m, pos, BIG_I), axis=1, keepdims=True)
        iv = jnp.sum(jnp.where(pos == p, comb_i, 0), axis=1, keepdims=True)
        new_v.append(m)
        new_i.append(iv)
        comb_v = jnp.where(pos == p, INF_F, comb_v)

    pad_v = jnp.full((Q, 8 - KNN), INF_F, jnp.float32)
    pad_i = jnp.zeros((Q, 8 - KNN), jnp.int32)
    top_v = jnp.concatenate(new_v + [pad_v], axis=1)
    top_i = jnp.concatenate(new_i + [pad_i], axis=1)

    idx_ref[...] = top_i

    @pl.when(j < nblk - 1)
    def _():
        vals_ref[...] = top_v

    @pl.when(j == nblk - 1)
    def _():
        a2 = jnp.sum(a * a, axis=1, keepdims=True)
        d = jnp.sqrt(jnp.maximum(top_v + a2, 1e-12))
        lane = jax.lax.broadcasted_iota(jnp.int32, (1, 8), 1)
        w = jnp.where(lane < KNN, 1.0 / (d + 1e-6), 0.0)
        w = w / jnp.sum(w, axis=1, keepdims=True)
        vals_ref[...] = w


def _run_topk(emb, refT_pad, nblk, blk, k_total):
    return pl.pallas_call(
        functools.partial(_topk_kernel, nblk=nblk, blk=blk, k_total=k_total),
        grid=(nblk,),
        in_specs=[
            pl.BlockSpec((Q, D), lambda j: (0, 0)),
            pl.BlockSpec((D, blk), lambda j: (0, j)),
        ],
        out_specs=[
            pl.BlockSpec((Q, 8), lambda j: (0, 0)),
            pl.BlockSpec((Q, 8), lambda j: (0, 0)),
        ],
        out_shape=[
            jax.ShapeDtypeStruct((Q, 8), jnp.float32),
            jax.ShapeDtypeStruct((Q, 8), jnp.int32),
        ],
        compiler_params=pltpu.CompilerParams(
            dimension_semantics=("arbitrary",)),
    )(emb, refT_pad)


NC, NS = 2, 16
NW = NC * NS
QPW = Q // NW
RPW = QPW * KNN


def _sc_gather_kernel(idx_hbm, w_hbm, aux_hbm, out_hbm,
                      idx_v, rows_v, w_v, out_v, sem):
    wid = lax.axis_index("s") * NC + lax.axis_index("c")
    base = wid * RPW
    pltpu.sync_copy(idx_hbm.at[pl.ds(base, RPW)], idx_v)
    pltpu.sync_copy(w_hbm.at[pl.ds(base, RPW)], w_v)
    half = RPW // 2
    cp1 = pltpu.async_copy(aux_hbm.at[idx_v.at[pl.ds(0, half)]],
                           rows_v.at[pl.ds(0, half)], sem)
    cp2 = pltpu.async_copy(aux_hbm.at[idx_v.at[pl.ds(half, half)]],
                           rows_v.at[pl.ds(half, half)], sem)
    cp1.wait()
    cp2.wait()
    for q in range(QPW):
        for dd in range(D_AUX // 16):
            sl = pl.ds(dd * 16, 16)
            acc = rows_v[q * KNN, sl] * w_v[q * KNN, sl]
            for t in range(1, KNN):
                acc = acc + rows_v[q * KNN + t, sl] * w_v[q * KNN + t, sl]
            out_v[q, sl] = acc
    pltpu.sync_copy(out_v, out_hbm.at[pl.ds(wid * QPW, QPW)])


def _run_sc_gather(idx_flat, w_rows, aux):
    mesh = plsc.VectorSubcoreMesh(core_axis_name="c", subcore_axis_name="s")
    f = functools.partial(
        pl.kernel,
        out_type=jax.ShapeDtypeStruct((Q, D_AUX), jnp.float32),
        mesh=mesh,
        scratch_types=[
            pltpu.VMEM((RPW,), jnp.int32),
            pltpu.VMEM((RPW, D_AUX), jnp.float32),
            pltpu.VMEM((RPW, D_AUX), jnp.float32),
            pltpu.VMEM((QPW, D_AUX), jnp.float32),
            pltpu.SemaphoreType.DMA,
        ],
        compiler_params=pltpu.CompilerParams(use_tc_tiling_on_sc=False),
    )(_sc_gather_kernel)
    return f(idx_flat, w_rows, aux)


def kernel(embedding_features, reference_embeddings, auxiliary_features):
    emb = embedding_features.reshape(Q, D)
    ref = reference_embeddings.reshape(-1, D)
    k_total = ref.shape[0]

    blk = 1024
    nblk = (k_total + blk - 1) // blk
    kpad = nblk * blk
    refT = ref.T
    refT_pad = jnp.pad(refT, ((0, 0), (0, kpad - k_total)))

    w8, idx8 = _run_topk(emb, refT_pad, nblk, blk, k_total)

    idx_flat = idx8[:, :KNN].reshape(-1)
    w_flat = w8[:, :KNN].reshape(-1)
    w_rows = jnp.broadcast_to(w_flat[:, None], (Q * KNN, D_AUX))

    aux = auxiliary_features.reshape(-1, D_AUX)
    return _run_sc_gather(idx_flat, w_rows, aux)


# speedup vs baseline: 2.8482x; 1.5962x over previous
import functools

import jax
import jax.numpy as jnp
from jax import lax
from jax.experimental import pallas as pl
from jax.experimental.pallas import tpu as pltpu
from jax.experimental.pallas import tpu_sc as plsc

Q = 1024
D = 16
D_AUX = 64
KNN = 5

INF_F = float("inf")
BIG_I = 2**30


MAXI = 2**31 - 1


def _topk_kernel(a_ref, bT_ref, w_ref, idx_ref, state_ref, *, nblk, blk,
                 k_total):
    j = pl.program_id(0)

    a = a_ref[...]
    bT = bT_ref[...]
    b2 = jnp.sum(bT * bT, axis=0, keepdims=True)
    a2 = jnp.sum(a * a, axis=1, keepdims=True)
    ab = lax.dot_general(a, bT, (((1,), (0,)), ((), ())),
                         preferred_element_type=jnp.float32)
    d2 = (a2 + b2) - 2.0 * ab
    d2 = jnp.maximum(d2, 0.0)

    col = jax.lax.broadcasted_iota(jnp.int32, (1, blk), 1)
    mrow = jax.lax.shift_right_logical(col, 7)
    bits = jax.lax.bitcast_convert_type(d2, jnp.int32)
    packed = jax.lax.bitwise_or(jax.lax.bitwise_and(bits, ~15), mrow)
    packed = jnp.where(col + j * blk < k_total, packed, MAXI)

    half = blk // 2
    v1 = jnp.minimum(packed[:, :half], packed[:, half:])
    v2 = jnp.maximum(packed[:, :half], packed[:, half:])
    while half > 128:
        half //= 2
        a1, b1 = v1[:, :half], v1[:, half:]
        a2_, b2_ = v2[:, :half], v2[:, half:]
        v1 = jnp.minimum(a1, b1)
        v2 = jnp.minimum(jnp.maximum(a1, b1), jnp.minimum(a2_, b2_))

    pool = jnp.concatenate([v1, v2], axis=1)
    ppos = jax.lax.broadcasted_iota(jnp.int32, (1, 256), 1)

    blk_v, blk_i = [], []
    for _ in range(KNN):
        m = jnp.min(pool, axis=1, keepdims=True)
        p = jnp.min(jnp.where(pool == m, ppos, BIG_I), axis=1, keepdims=True)
        pool = jnp.where(ppos == p, MAXI, pool)
        lane_c = jax.lax.bitwise_and(p, 127)
        colw = j * blk + lane_c + 128 * jax.lax.bitwise_and(m, 15)
        blk_v.append(m)
        blk_i.append(colw)

    first = j == 0
    cur_v = jnp.where(first, MAXI, state_ref[...])
    cur_i = jnp.where(first, 0, idx_ref[...])

    comb_v = jnp.concatenate([cur_v] + blk_v, axis=1)
    comb_i = jnp.concatenate([cur_i] + blk_i, axis=1)
    pos = jax.lax.broadcasted_iota(jnp.int32, (1, 13), 1)

    new_v, new_i = [], []
    for _ in range(KNN):
        m = jnp.min(comb_v, axis=1, keepdims=True)
        p = jnp.min(jnp.where(comb_v == m, pos, BIG_I), axis=1, keepdims=True)
        iv = jnp.sum(jnp.where(pos == p, comb_i, 0), axis=1, keepdims=True)
        new_v.append(m)
        new_i.append(iv)
        comb_v = jnp.where(pos == p, MAXI, comb_v)

    pad_v = jnp.full((Q, 8 - KNN), MAXI, jnp.int32)
    pad_i = jnp.zeros((Q, 8 - KNN), jnp.int32)
    top_v = jnp.concatenate(new_v + [pad_v], axis=1)
    top_i = jnp.concatenate(new_i + [pad_i], axis=1)

    idx_ref[...] = top_i
    state_ref[...] = top_v

    @pl.when(j == nblk - 1)
    def _():
        d2w = jax.lax.bitcast_convert_type(
            jax.lax.bitwise_and(top_v, ~15), jnp.float32)
        d = jnp.sqrt(jnp.maximum(d2w, 1e-12))
        lane = jax.lax.broadcasted_iota(jnp.int32, (1, 8), 1)
        w = jnp.where(lane < KNN, 1.0 / (d + 1e-6), 0.0)
        w_ref[...] = w / jnp.sum(w, axis=1, keepdims=True)


def _run_topk(emb, refT_pad, nblk, blk, k_total):
    return pl.pallas_call(
        functools.partial(_topk_kernel, nblk=nblk, blk=blk, k_total=k_total),
        grid=(nblk,),
        in_specs=[
            pl.BlockSpec((Q, D), lambda j: (0, 0)),
            pl.BlockSpec((D, blk), lambda j: (0, j)),
        ],
        out_specs=[
            pl.BlockSpec((Q, 8), lambda j: (0, 0)),
            pl.BlockSpec((Q, 8), lambda j: (0, 0)),
            pl.BlockSpec((Q, 8), lambda j: (0, 0)),
        ],
        out_shape=[
            jax.ShapeDtypeStruct((Q, 8), jnp.float32),
            jax.ShapeDtypeStruct((Q, 8), jnp.int32),
            jax.ShapeDtypeStruct((Q, 8), jnp.int32),
        ],
        compiler_params=pltpu.CompilerParams(
            dimension_semantics=("arbitrary",)),
    )(emb, refT_pad)


NC, NS = 2, 16
NW = NC * NS
QPW = Q // NW
RPW = QPW * KNN


def _sc_gather_kernel(idx_hbm, w_hbm, aux_hbm, out_hbm,
                      idx_v, rows_v, w_v, out_v, sem):
    wid = lax.axis_index("s") * NC + lax.axis_index("c")
    base = wid * RPW
    pltpu.sync_copy(idx_hbm.at[pl.ds(base, RPW)], idx_v)
    pltpu.sync_copy(w_hbm.at[pl.ds(base, RPW)], w_v)
    half = RPW // 2
    cp1 = pltpu.async_copy(aux_hbm.at[idx_v.at[pl.ds(0, half)]],
                           rows_v.at[pl.ds(0, half)], sem)
    cp2 = pltpu.async_copy(aux_hbm.at[idx_v.at[pl.ds(half, half)]],
                           rows_v.at[pl.ds(half, half)], sem)
    cp1.wait()
    cp2.wait()
    for q in range(QPW):
        for dd in range(D_AUX // 16):
            sl = pl.ds(dd * 16, 16)
            acc = rows_v[q * KNN, sl] * w_v[q * KNN, sl]
            for t in range(1, KNN):
                acc = acc + rows_v[q * KNN + t, sl] * w_v[q * KNN + t, sl]
            out_v[q, sl] = acc
    pltpu.sync_copy(out_v, out_hbm.at[pl.ds(wid * QPW, QPW)])


def _run_sc_gather(idx_flat, w_rows, aux):
    mesh = plsc.VectorSubcoreMesh(core_axis_name="c", subcore_axis_name="s")
    f = functools.partial(
        pl.kernel,
        out_type=jax.ShapeDtypeStruct((Q, D_AUX), jnp.float32),
        mesh=mesh,
        scratch_types=[
            pltpu.VMEM((RPW,), jnp.int32),
            pltpu.VMEM((RPW, D_AUX), jnp.float32),
            pltpu.VMEM((RPW, D_AUX), jnp.float32),
            pltpu.VMEM((QPW, D_AUX), jnp.float32),
            pltpu.SemaphoreType.DMA,
        ],
        compiler_params=pltpu.CompilerParams(use_tc_tiling_on_sc=False),
    )(_sc_gather_kernel)
    return f(idx_flat, w_rows, aux)


def kernel(embedding_features, reference_embeddings, auxiliary_features):
    emb = embedding_features.reshape(Q, D)
    ref = reference_embeddings.reshape(-1, D)
    k_total = ref.shape[0]

    blk = 2048
    nblk = (k_total + blk - 1) // blk
    kpad = nblk * blk
    refT = ref.T
    refT_pad = jnp.pad(refT, ((0, 0), (0, kpad - k_total)))

    w8, idx8, _ = _run_topk(emb, refT_pad, nblk, blk, k_total)

    idx_flat = idx8[:, :KNN].reshape(-1)
    w_flat = w8[:, :KNN].reshape(-1)
    w_rows = jnp.broadcast_to(w_flat[:, None], (Q * KNN, D_AUX))

    aux = auxiliary_features.reshape(-1, D_AUX)
    return _run_sc_gather(idx_flat, w_rows, aux)


# speedup vs baseline: 6.0006x; 2.1068x over previous
import functools

import jax
import jax.numpy as jnp
from jax import lax
from jax.experimental import pallas as pl
from jax.experimental.pallas import tpu as pltpu
from jax.experimental.pallas import tpu_sc as plsc

Q = 1024
D = 16
D_AUX = 64
KNN = 5

INF_F = float("inf")
BIG_I = 2**30


MAXI = 2**31 - 1


def _ce(bv, bi, rv, ri):
    c = bv < rv
    return (jnp.where(c, bv, rv), jnp.where(c, bi, ri),
            jnp.where(c, rv, bv), jnp.where(c, ri, bi))


def _topk_kernel(a_ref, bT_ref, w_ref, idx_ref,
                 r1v, r1i, r2v, r2i, r3v, r3i, *, nblk, blk, k_total):
    j = pl.program_id(0)

    a = a_ref[...]
    bT = bT_ref[...]
    b2 = jnp.sum(bT * bT, axis=0, keepdims=True)
    a2 = jnp.sum(a * a, axis=1, keepdims=True)
    ab2 = lax.dot_general(a * -2.0, bT, (((1,), (0,)), ((), ())),
                          preferred_element_type=jnp.float32)
    d2 = jnp.maximum((a2 + b2) + ab2, 0.0)

    col = jax.lax.broadcasted_iota(jnp.int32, (1, blk), 1)
    mrow = jax.lax.shift_right_logical(col, 7)
    bits = jax.lax.bitcast_convert_type(d2, jnp.int32)
    packed = jax.lax.bitwise_or(jax.lax.bitwise_and(bits, ~15), mrow)
    packed = jnp.where(col + j * blk < k_total, packed, MAXI)

    half = blk // 2
    v1 = jnp.minimum(packed[:, :half], packed[:, half:])
    v2 = jnp.maximum(packed[:, :half], packed[:, half:])
    while half > 128:
        half //= 2
        a1, b1 = v1[:, :half], v1[:, half:]
        a2_, b2_ = v2[:, :half], v2[:, half:]
        v1 = jnp.minimum(a1, b1)
        v2 = jnp.minimum(jnp.maximum(a1, b1), jnp.minimum(a2_, b2_))

    lane_c = jax.lax.broadcasted_iota(jnp.int32, (1, 128), 1) + j * blk
    bi1 = lane_c + 128 * jax.lax.bitwise_and(v1, 15)
    bi2 = lane_c + 128 * jax.lax.bitwise_and(v2, 15)

    first = j == 0
    c1v = jnp.where(first, MAXI, r1v[...])
    c1i = jnp.where(first, BIG_I, r1i[...])
    c2v = jnp.where(first, MAXI, r2v[...])
    c2i = jnp.where(first, BIG_I, r2i[...])
    c3v = jnp.where(first, MAXI, r3v[...])
    c3i = jnp.where(first, BIG_I, r3i[...])

    for bv, bi in ((v1, bi1), (v2, bi2)):
        c1v, c1i, x, xi = _ce(bv, bi, c1v, c1i)
        c2v, c2i, y, yi = _ce(x, xi, c2v, c2i)
        c = y < c3v
        c3v = jnp.where(c, y, c3v)
        c3i = jnp.where(c, yi, c3i)

    r1v[...], r1i[...] = c1v, c1i
    r2v[...], r2i[...] = c2v, c2i
    r3v[...], r3i[...] = c3v, c3i

    @pl.when(j == nblk - 1)
    def _():
        pool = jnp.concatenate([c1v, c2v, c3v], axis=1)
        pooli = jnp.concatenate([c1i, c2i, c3i], axis=1)
        new_v, new_i = [], []
        for _ in range(KNN):
            m = jnp.min(pool, axis=1, keepdims=True)
            eq = pool == m
            p = jnp.min(jnp.where(eq, pooli, BIG_I), axis=1, keepdims=True)
            pool = jnp.where(eq & (pooli == p), MAXI, pool)
            new_v.append(m)
            new_i.append(p)
        pad_v = jnp.full((Q, 8 - KNN), MAXI, jnp.int32)
        pad_i = jnp.zeros((Q, 8 - KNN), jnp.int32)
        top_v = jnp.concatenate(new_v + [pad_v], axis=1)
        top_i = jnp.concatenate(new_i + [pad_i], axis=1)

        idx_ref[...] = top_i
        d2w = jax.lax.bitcast_convert_type(
            jax.lax.bitwise_and(top_v, ~15), jnp.float32)
        d = jnp.sqrt(jnp.maximum(d2w, 1e-12))
        lane = jax.lax.broadcasted_iota(jnp.int32, (1, 8), 1)
        w = jnp.where(lane < KNN, 1.0 / (d + 1e-6), 0.0)
        w_ref[...] = w / jnp.sum(w, axis=1, keepdims=True)


def _run_topk(emb, refT_pad, nblk, blk, k_total):
    return pl.pallas_call(
        functools.partial(_topk_kernel, nblk=nblk, blk=blk, k_total=k_total),
        grid=(nblk,),
        in_specs=[
            pl.BlockSpec((Q, D), lambda j: (0, 0)),
            pl.BlockSpec((D, blk), lambda j: (0, j)),
        ],
        out_specs=[
            pl.BlockSpec((Q, 8), lambda j: (0, 0)),
            pl.BlockSpec((Q, 8), lambda j: (0, 0)),
        ],
        out_shape=[
            jax.ShapeDtypeStruct((Q, 8), jnp.float32),
            jax.ShapeDtypeStruct((Q, 8), jnp.int32),
        ],
        scratch_shapes=[pltpu.VMEM((Q, 128), jnp.int32)] * 6,
        compiler_params=pltpu.CompilerParams(
            dimension_semantics=("arbitrary",)),
    )(emb, refT_pad)


NC, NS = 2, 16
NW = NC * NS
QPW = Q // NW
RPW = QPW * KNN


def _sc_gather_kernel(idx_hbm, w_hbm, aux_hbm, out_hbm,
                      idx_v, rows_v, w_v, out_v, sem):
    wid = lax.axis_index("s") * NC + lax.axis_index("c")
    base = wid * RPW
    pltpu.sync_copy(idx_hbm.at[pl.ds(base, RPW)], idx_v)
    pltpu.sync_copy(w_hbm.at[pl.ds(base, RPW)], w_v)
    half = RPW // 2
    cp1 = pltpu.async_copy(aux_hbm.at[idx_v.at[pl.ds(0, half)]],
                           rows_v.at[pl.ds(0, half)], sem)
    cp2 = pltpu.async_copy(aux_hbm.at[idx_v.at[pl.ds(half, half)]],
                           rows_v.at[pl.ds(half, half)], sem)
    cp1.wait()
    cp2.wait()
    for q in range(QPW):
        for dd in range(D_AUX // 16):
            sl = pl.ds(dd * 16, 16)
            acc = rows_v[q * KNN, sl] * w_v[q * KNN, sl]
            for t in range(1, KNN):
                acc = acc + rows_v[q * KNN + t, sl] * w_v[q * KNN + t, sl]
            out_v[q, sl] = acc
    pltpu.sync_copy(out_v, out_hbm.at[pl.ds(wid * QPW, QPW)])


def _run_sc_gather(idx_flat, w_rows, aux):
    mesh = plsc.VectorSubcoreMesh(core_axis_name="c", subcore_axis_name="s")
    f = functools.partial(
        pl.kernel,
        out_type=jax.ShapeDtypeStruct((Q, D_AUX), jnp.float32),
        mesh=mesh,
        scratch_types=[
            pltpu.VMEM((RPW,), jnp.int32),
            pltpu.VMEM((RPW, D_AUX), jnp.float32),
            pltpu.VMEM((RPW, D_AUX), jnp.float32),
            pltpu.VMEM((QPW, D_AUX), jnp.float32),
            pltpu.SemaphoreType.DMA,
        ],
        compiler_params=pltpu.CompilerParams(use_tc_tiling_on_sc=False),
    )(_sc_gather_kernel)
    return f(idx_flat, w_rows, aux)


def kernel(embedding_features, reference_embeddings, auxiliary_features):
    emb = embedding_features.reshape(Q, D)
    ref = reference_embeddings.reshape(-1, D)
    k_total = ref.shape[0]

    blk = 2048
    nblk = (k_total + blk - 1) // blk
    kpad = nblk * blk
    refT = ref.T
    refT_pad = jnp.pad(refT, ((0, 0), (0, kpad - k_total)))

    w8, idx8 = _run_topk(emb, refT_pad, nblk, blk, k_total)

    idx_flat = idx8[:, :KNN].reshape(-1)
    w_flat = w8[:, :KNN].reshape(-1)
    w_rows = jnp.broadcast_to(w_flat[:, None], (Q * KNN, D_AUX))

    aux = auxiliary_features.reshape(-1, D_AUX)
    return _run_sc_gather(idx_flat, w_rows, aux)


# speedup vs baseline: 6.3102x; 1.0516x over previous
import functools

import jax
import jax.numpy as jnp
from jax import lax
from jax.experimental import pallas as pl
from jax.experimental.pallas import tpu as pltpu
from jax.experimental.pallas import tpu_sc as plsc

Q = 1024
D = 16
D_AUX = 64
KNN = 5

INF_F = float("inf")
BIG_I = 2**30


MAXI = 2**31 - 1


def _ce(bv, bi, rv, ri):
    c = bv < rv
    return (jnp.where(c, bv, rv), jnp.where(c, bi, ri),
            jnp.where(c, rv, bv), jnp.where(c, ri, bi))


def _topk_kernel(a_ref, bT_ref, w_ref, idx_ref,
                 r1v, r1i, r2v, r2i, r3v, r3i, *, nblk, blk, k_total):
    j = pl.program_id(0)

    a = a_ref[...]
    bT = bT_ref[...]
    b2 = jnp.sum(bT * bT, axis=0, keepdims=True)
    a2 = jnp.sum(a * a, axis=1, keepdims=True)
    ab2 = lax.dot_general(a * -2.0, bT, (((1,), (0,)), ((), ())),
                          preferred_element_type=jnp.float32)
    d2 = jnp.maximum((a2 + b2) + ab2, 0.0)

    pmask = blk // 128 - 1
    col = jax.lax.broadcasted_iota(jnp.int32, (1, blk), 1)
    mrow = jax.lax.shift_right_logical(col, 7)
    bits = jax.lax.bitcast_convert_type(d2, jnp.int32)
    packed = jax.lax.bitwise_or(jax.lax.bitwise_and(bits, ~pmask), mrow)
    packed = jnp.where(col + j * blk < k_total, packed, MAXI)

    half = blk // 2
    v1 = jnp.minimum(packed[:, :half], packed[:, half:])
    v2 = jnp.maximum(packed[:, :half], packed[:, half:])
    while half > 128:
        half //= 2
        a1, b1 = v1[:, :half], v1[:, half:]
        a2_, b2_ = v2[:, :half], v2[:, half:]
        v1 = jnp.minimum(a1, b1)
        v2 = jnp.minimum(jnp.maximum(a1, b1), jnp.minimum(a2_, b2_))

    lane_c = jax.lax.broadcasted_iota(jnp.int32, (1, 128), 1) + j * blk
    bi1 = lane_c + 128 * jax.lax.bitwise_and(v1, pmask)
    bi2 = lane_c + 128 * jax.lax.bitwise_and(v2, pmask)

    first = j == 0
    c1v = jnp.where(first, MAXI, r1v[...])
    c1i = jnp.where(first, BIG_I, r1i[...])
    c2v = jnp.where(first, MAXI, r2v[...])
    c2i = jnp.where(first, BIG_I, r2i[...])
    c3v = jnp.where(first, MAXI, r3v[...])
    c3i = jnp.where(first, BIG_I, r3i[...])

    for bv, bi in ((v1, bi1), (v2, bi2)):
        c1v, c1i, x, xi = _ce(bv, bi, c1v, c1i)
        c2v, c2i, y, yi = _ce(x, xi, c2v, c2i)
        c = y < c3v
        c3v = jnp.where(c, y, c3v)
        c3i = jnp.where(c, yi, c3i)

    r1v[...], r1i[...] = c1v, c1i
    r2v[...], r2i[...] = c2v, c2i
    r3v[...], r3i[...] = c3v, c3i

    @pl.when(j == nblk - 1)
    def _():
        pool = jnp.concatenate([c1v, c2v, c3v], axis=1)
        pooli = jnp.concatenate([c1i, c2i, c3i], axis=1)
        new_v, new_i = [], []
        for _ in range(KNN):
            m = jnp.min(pool, axis=1, keepdims=True)
            eq = pool == m
            p = jnp.min(jnp.where(eq, pooli, BIG_I), axis=1, keepdims=True)
            pool = jnp.where(eq & (pooli == p), MAXI, pool)
            new_v.append(m)
            new_i.append(p)
        pad_v = jnp.full((Q, 8 - KNN), MAXI, jnp.int32)
        pad_i = jnp.zeros((Q, 8 - KNN), jnp.int32)
        top_v = jnp.concatenate(new_v + [pad_v], axis=1)
        top_i = jnp.concatenate(new_i + [pad_i], axis=1)

        idx_ref[...] = top_i
        d2w = jax.lax.bitcast_convert_type(
            jax.lax.bitwise_and(top_v, ~pmask), jnp.float32)
        d = jnp.sqrt(jnp.maximum(d2w, 1e-12))
        lane = jax.lax.broadcasted_iota(jnp.int32, (1, 8), 1)
        w = jnp.where(lane < KNN, 1.0 / (d + 1e-6), 0.0)
        w_ref[...] = w / jnp.sum(w, axis=1, keepdims=True)


def _run_topk(emb, refT_pad, nblk, blk, k_total):
    return pl.pallas_call(
        functools.partial(_topk_kernel, nblk=nblk, blk=blk, k_total=k_total),
        grid=(nblk,),
        in_specs=[
            pl.BlockSpec((Q, D), lambda j: (0, 0)),
            pl.BlockSpec((D, blk), lambda j: (0, j)),
        ],
        out_specs=[
            pl.BlockSpec((Q, 8), lambda j: (0, 0)),
            pl.BlockSpec((Q, 8), lambda j: (0, 0)),
        ],
        out_shape=[
            jax.ShapeDtypeStruct((Q, 8), jnp.float32),
            jax.ShapeDtypeStruct((Q, 8), jnp.int32),
        ],
        scratch_shapes=[pltpu.VMEM((Q, 128), jnp.int32)] * 6,
        compiler_params=pltpu.CompilerParams(
            dimension_semantics=("arbitrary",)),
    )(emb, refT_pad)


NC, NS = 2, 16
NW = NC * NS
QPW = Q // NW
RPW = QPW * KNN


def _sc_gather_kernel(idx_hbm, w_hbm, aux_hbm, out_hbm,
                      idx_v, rows_v, w_v, out_v, sem):
    wid = lax.axis_index("s") * NC + lax.axis_index("c")
    base = wid * RPW
    pltpu.sync_copy(idx_hbm.at[pl.ds(base, RPW)], idx_v)
    pltpu.sync_copy(w_hbm.at[pl.ds(base, RPW)], w_v)
    half = RPW // 2
    cp1 = pltpu.async_copy(aux_hbm.at[idx_v.at[pl.ds(0, half)]],
                           rows_v.at[pl.ds(0, half)], sem)
    cp2 = pltpu.async_copy(aux_hbm.at[idx_v.at[pl.ds(half, half)]],
                           rows_v.at[pl.ds(half, half)], sem)
    cp1.wait()
    cp2.wait()
    for q in range(QPW):
        for dd in range(D_AUX // 16):
            sl = pl.ds(dd * 16, 16)
            acc = rows_v[q * KNN, sl] * w_v[q * KNN, sl]
            for t in range(1, KNN):
                acc = acc + rows_v[q * KNN + t, sl] * w_v[q * KNN + t, sl]
            out_v[q, sl] = acc
    pltpu.sync_copy(out_v, out_hbm.at[pl.ds(wid * QPW, QPW)])


def _run_sc_gather(idx_flat, w_rows, aux):
    mesh = plsc.VectorSubcoreMesh(core_axis_name="c", subcore_axis_name="s")
    f = functools.partial(
        pl.kernel,
        out_type=jax.ShapeDtypeStruct((Q, D_AUX), jnp.float32),
        mesh=mesh,
        scratch_types=[
            pltpu.VMEM((RPW,), jnp.int32),
            pltpu.VMEM((RPW, D_AUX), jnp.float32),
            pltpu.VMEM((RPW, D_AUX), jnp.float32),
            pltpu.VMEM((QPW, D_AUX), jnp.float32),
            pltpu.SemaphoreType.DMA,
        ],
        compiler_params=pltpu.CompilerParams(use_tc_tiling_on_sc=False),
    )(_sc_gather_kernel)
    return f(idx_flat, w_rows, aux)


def kernel(embedding_features, reference_embeddings, auxiliary_features):
    emb = embedding_features.reshape(Q, D)
    ref = reference_embeddings.reshape(-1, D)
    k_total = ref.shape[0]

    blk = 4096
    nblk = (k_total + blk - 1) // blk
    kpad = nblk * blk
    refT = ref.T
    refT_pad = jnp.pad(refT, ((0, 0), (0, kpad - k_total)))

    w8, idx8 = _run_topk(emb, refT_pad, nblk, blk, k_total)

    idx_flat = idx8[:, :KNN].reshape(-1)
    w_flat = w8[:, :KNN].reshape(-1)
    w_rows = jnp.broadcast_to(w_flat[:, None], (Q * KNN, D_AUX))

    aux = auxiliary_features.reshape(-1, D_AUX)
    return _run_sc_gather(idx_flat, w_rows, aux)


# speedup vs baseline: 6.4448x; 1.0213x over previous
import functools

import jax
import jax.numpy as jnp
from jax import lax
from jax.experimental import pallas as pl
from jax.experimental.pallas import tpu as pltpu
from jax.experimental.pallas import tpu_sc as plsc

Q = 1024
D = 16
D_AUX = 64
KNN = 5

INF_F = float("inf")
BIG_I = 2**30


MAXI = 2**31 - 1


def _ce(bv, bi, rv, ri):
    c = bv < rv
    return (jnp.where(c, bv, rv), jnp.where(c, bi, ri),
            jnp.where(c, rv, bv), jnp.where(c, ri, bi))


def _topk_kernel(a_ref, bT_ref, w_ref, idx_ref,
                 r1v, r1i, r2v, r2i, r3v, r3i, *, nblk, blk, k_total):
    j = pl.program_id(0)

    a = a_ref[...]
    bT = bT_ref[...]
    b2 = jnp.sum(bT * bT, axis=0, keepdims=True)
    a2 = jnp.sum(a * a, axis=1, keepdims=True)
    ab2 = lax.dot_general(a * -2.0, bT, (((1,), (0,)), ((), ())),
                          preferred_element_type=jnp.float32)
    d2 = jnp.maximum((a2 + b2) + ab2, 0.0)

    pmask = blk // 128 - 1
    col = jax.lax.broadcasted_iota(jnp.int32, (1, blk), 1)
    mrow = jax.lax.shift_right_logical(col, 7)
    bits = jax.lax.bitcast_convert_type(d2, jnp.int32)
    packed = jax.lax.bitwise_or(jax.lax.bitwise_and(bits, ~pmask), mrow)
    packed = jnp.where(col + j * blk < k_total, packed, MAXI)

    half = blk // 2
    v1 = jnp.minimum(packed[:, :half], packed[:, half:])
    v2 = jnp.maximum(packed[:, :half], packed[:, half:])
    while half > 128:
        half //= 2
        a1, b1 = v1[:, :half], v1[:, half:]
        a2_, b2_ = v2[:, :half], v2[:, half:]
        v1 = jnp.minimum(a1, b1)
        v2 = jnp.minimum(jnp.maximum(a1, b1), jnp.minimum(a2_, b2_))

    lane_c = jax.lax.broadcasted_iota(jnp.int32, (1, 128), 1) + j * blk
    bi1 = lane_c + 128 * jax.lax.bitwise_and(v1, pmask)
    bi2 = lane_c + 128 * jax.lax.bitwise_and(v2, pmask)

    first = j == 0
    c1v = jnp.where(first, MAXI, r1v[...])
    c1i = jnp.where(first, BIG_I, r1i[...])
    c2v = jnp.where(first, MAXI, r2v[...])
    c2i = jnp.where(first, BIG_I, r2i[...])
    c3v = jnp.where(first, MAXI, r3v[...])
    c3i = jnp.where(first, BIG_I, r3i[...])

    for bv, bi in ((v1, bi1), (v2, bi2)):
        c1v, c1i, x, xi = _ce(bv, bi, c1v, c1i)
        c2v, c2i, y, yi = _ce(x, xi, c2v, c2i)
        c = y < c3v
        c3v = jnp.where(c, y, c3v)
        c3i = jnp.where(c, yi, c3i)

    r1v[...], r1i[...] = c1v, c1i
    r2v[...], r2i[...] = c2v, c2i
    r3v[...], r3i[...] = c3v, c3i

    @pl.when(j == nblk - 1)
    def _():
        pool = jnp.concatenate([c1v, c2v, c3v], axis=1)
        pooli = jnp.concatenate([c1i, c2i, c3i], axis=1)
        new_v, new_i = [], []
        for _ in range(KNN):
            m = jnp.min(pool, axis=1, keepdims=True)
            eq = pool == m
            p = jnp.min(jnp.where(eq, pooli, BIG_I), axis=1, keepdims=True)
            pool = jnp.where(eq & (pooli == p), MAXI, pool)
            new_v.append(m)
            new_i.append(p)
        pad_v = jnp.full((Q, 8 - KNN), MAXI, jnp.int32)
        pad_i = jnp.zeros((Q, 8 - KNN), jnp.int32)
        top_v = jnp.concatenate(new_v + [pad_v], axis=1)
        top_i = jnp.concatenate(new_i + [pad_i], axis=1)

        idx_ref[...] = top_i
        d2w = jax.lax.bitcast_convert_type(
            jax.lax.bitwise_and(top_v, ~pmask), jnp.float32)
        d = jnp.sqrt(jnp.maximum(d2w, 1e-12))
        lane = jax.lax.broadcasted_iota(jnp.int32, (1, 8), 1)
        w = jnp.where(lane < KNN, 1.0 / (d + 1e-6), 0.0)
        w_ref[...] = w / jnp.sum(w, axis=1, keepdims=True)


def _run_topk(emb, refT_pad, nblk, blk, k_total):
    return pl.pallas_call(
        functools.partial(_topk_kernel, nblk=nblk, blk=blk, k_total=k_total),
        grid=(nblk,),
        in_specs=[
            pl.BlockSpec((Q, D), lambda j: (0, 0)),
            pl.BlockSpec((D, blk), lambda j: (0, j)),
        ],
        out_specs=[
            pl.BlockSpec((Q, 8), lambda j: (0, 0)),
            pl.BlockSpec((Q, 8), lambda j: (0, 0)),
        ],
        out_shape=[
            jax.ShapeDtypeStruct((Q, 8), jnp.float32),
            jax.ShapeDtypeStruct((Q, 8), jnp.int32),
        ],
        scratch_shapes=[pltpu.VMEM((Q, 128), jnp.int32)] * 6,
        compiler_params=pltpu.CompilerParams(
            dimension_semantics=("arbitrary",)),
    )(emb, refT_pad)


NC, NS = 2, 16
NW = NC * NS
QPW = Q // NW
RPW = QPW * KNN


def _sc_gather_kernel(idx_hbm, w_hbm, aux_hbm, out_hbm,
                      idx_v, rows_v, w_v, out_v, sem):
    wid = lax.axis_index("s") * NC + lax.axis_index("c")
    base = wid * RPW
    pltpu.sync_copy(idx_hbm.at[pl.ds(base, RPW)], idx_v)
    pltpu.sync_copy(w_hbm.at[pl.ds(base, RPW)], w_v)
    half = RPW // 2
    cp1 = pltpu.async_copy(aux_hbm.at[idx_v.at[pl.ds(0, half)]],
                           rows_v.at[pl.ds(0, half)], sem)
    cp2 = pltpu.async_copy(aux_hbm.at[idx_v.at[pl.ds(half, half)]],
                           rows_v.at[pl.ds(half, half)], sem)
    cp1.wait()
    cp2.wait()
    for q in range(QPW):
        for dd in range(D_AUX // 16):
            sl = pl.ds(dd * 16, 16)
            acc = rows_v[q * KNN, sl] * w_v[q * KNN, sl]
            for t in range(1, KNN):
                acc = acc + rows_v[q * KNN + t, sl] * w_v[q * KNN + t, sl]
            out_v[q, sl] = acc
    pltpu.sync_copy(out_v, out_hbm.at[pl.ds(wid * QPW, QPW)])


def _run_sc_gather(idx_flat, w_rows, aux):
    mesh = plsc.VectorSubcoreMesh(core_axis_name="c", subcore_axis_name="s")
    f = functools.partial(
        pl.kernel,
        out_type=jax.ShapeDtypeStruct((Q, D_AUX), jnp.float32),
        mesh=mesh,
        scratch_types=[
            pltpu.VMEM((RPW,), jnp.int32),
            pltpu.VMEM((RPW, 128), jnp.float32),
            pltpu.VMEM((RPW, D_AUX), jnp.float32),
            pltpu.VMEM((QPW, D_AUX), jnp.float32),
            pltpu.SemaphoreType.DMA,
        ],
    )(_sc_gather_kernel)
    return f(idx_flat, w_rows, aux)


def kernel(embedding_features, reference_embeddings, auxiliary_features):
    emb = embedding_features.reshape(Q, D)
    ref = reference_embeddings.reshape(-1, D)
    k_total = ref.shape[0]

    blk = 4096
    nblk = (k_total + blk - 1) // blk
    kpad = nblk * blk
    refT = ref.T
    refT_pad = jnp.pad(refT, ((0, 0), (0, kpad - k_total)))

    w8, idx8 = _run_topk(emb, refT_pad, nblk, blk, k_total)

    idx_flat = idx8[:, :KNN].reshape(-1)
    w_flat = w8[:, :KNN].reshape(-1)
    w_rows = jnp.broadcast_to(w_flat[:, None], (Q * KNN, D_AUX))

    aux = auxiliary_features.reshape(-1, D_AUX)
    aux_pad = jnp.pad(aux, ((0, 0), (0, 128 - D_AUX)))
    return _run_sc_gather(idx_flat, w_rows, aux_pad)


# speedup vs baseline: 7.0778x; 1.0982x over previous
import functools

import jax
import jax.numpy as jnp
from jax import lax
from jax.experimental import pallas as pl
from jax.experimental.pallas import tpu as pltpu
from jax.experimental.pallas import tpu_sc as plsc

Q = 1024
D = 16
D_AUX = 64
KNN = 5

INF_F = float("inf")
BIG_I = 2**30


MAXI = 2**31 - 1


def _ce(bv, bi, rv, ri):
    c = bv < rv
    return (jnp.where(c, bv, rv), jnp.where(c, bi, ri),
            jnp.where(c, rv, bv), jnp.where(c, ri, bi))


def _topk_kernel(a_ref, bT_ref, w_ref, idx_ref,
                 r1v, r1i, r2v, r2i, r3v, r3i, aug_ref, *, nblk, blk,
                 k_total):
    j = pl.program_id(0)

    a = a_ref[...]
    bT = bT_ref[...]
    b2 = jnp.sum(bT * bT, axis=0, keepdims=True)
    a2 = jnp.sum(a * a, axis=1, keepdims=True)

    ab2 = lax.dot_general(a * -2.0, bT, (((1,), (0,)), ((), ())),
                          preferred_element_type=jnp.float32)
    d2 = (a2 + b2) + ab2

    pmask = blk // 128 - 1
    col = jax.lax.broadcasted_iota(jnp.int32, (1, blk), 1)
    mrow = jax.lax.shift_right_logical(col, 7)
    bits = jax.lax.bitcast_convert_type(d2, jnp.int32)
    packed = jax.lax.bitwise_or(jax.lax.bitwise_and(bits, ~pmask), mrow)

    half = blk // 2
    v1 = jnp.minimum(packed[:, :half], packed[:, half:])
    v2 = jnp.maximum(packed[:, :half], packed[:, half:])
    while half > 128:
        half //= 2
        a1, b1 = v1[:, :half], v1[:, half:]
        a2_, b2_ = v2[:, :half], v2[:, half:]
        v1 = jnp.minimum(a1, b1)
        v2 = jnp.minimum(jnp.maximum(a1, b1), jnp.minimum(a2_, b2_))

    lane_c = jax.lax.broadcasted_iota(jnp.int32, (1, 128), 1) + j * blk
    bi1 = lane_c + 128 * jax.lax.bitwise_and(v1, pmask)
    bi2 = lane_c + 128 * jax.lax.bitwise_and(v2, pmask)

    first = j == 0
    c1v = jnp.where(first, MAXI, r1v[...])
    c1i = jnp.where(first, BIG_I, r1i[...])
    c2v = jnp.where(first, MAXI, r2v[...])
    c2i = jnp.where(first, BIG_I, r2i[...])
    c3v = jnp.where(first, MAXI, r3v[...])
    c3i = jnp.where(first, BIG_I, r3i[...])

    for bv, bi in ((v1, bi1), (v2, bi2)):
        c1v, c1i, x, xi = _ce(bv, bi, c1v, c1i)
        c2v, c2i, y, yi = _ce(x, xi, c2v, c2i)
        c = y < c3v
        c3v = jnp.where(c, y, c3v)
        c3i = jnp.where(c, yi, c3i)

    r1v[...], r1i[...] = c1v, c1i
    r2v[...], r2i[...] = c2v, c2i
    r3v[...], r3i[...] = c3v, c3i

    @pl.when(j == nblk - 1)
    def _():
        pool = jnp.concatenate([c1v, c2v, c3v], axis=1)
        pooli = jnp.concatenate([c1i, c2i, c3i], axis=1)
        new_v, new_i = [], []
        for _ in range(KNN):
            m = jnp.min(pool, axis=1, keepdims=True)
            eq = pool == m
            p = jnp.min(jnp.where(eq, pooli, BIG_I), axis=1, keepdims=True)
            pool = jnp.where(eq & (pooli == p), MAXI, pool)
            new_v.append(m)
            new_i.append(p)
        pad_v = jnp.full((Q, 8 - KNN), MAXI, jnp.int32)
        pad_i = jnp.zeros((Q, 8 - KNN), jnp.int32)
        top_v = jnp.concatenate(new_v + [pad_v], axis=1)
        top_i = jnp.concatenate(new_i + [pad_i], axis=1)

        idx_ref[...] = top_i
        d2w = jax.lax.bitcast_convert_type(
            jax.lax.bitwise_and(top_v, ~pmask), jnp.float32)
        d = jnp.sqrt(jnp.maximum(d2w, 1e-12))
        lane = jax.lax.broadcasted_iota(jnp.int32, (1, 8), 1)
        w = jnp.where(lane < KNN, 1.0 / (d + 1e-6), 0.0)
        w_ref[...] = w / jnp.sum(w, axis=1, keepdims=True)


def _run_topk(emb, refT_pad, nblk, blk, k_total):
    return pl.pallas_call(
        functools.partial(_topk_kernel, nblk=nblk, blk=blk, k_total=k_total),
        grid=(nblk,),
        in_specs=[
            pl.BlockSpec((Q, D), lambda j: (0, 0)),
            pl.BlockSpec((D, blk), lambda j: (0, j)),
        ],
        out_specs=[
            pl.BlockSpec((Q, 8), lambda j: (0, 0)),
            pl.BlockSpec((Q, 8), lambda j: (0, 0)),
        ],
        out_shape=[
            jax.ShapeDtypeStruct((Q, 8), jnp.float32),
            jax.ShapeDtypeStruct((Q, 8), jnp.int32),
        ],
        scratch_shapes=[pltpu.VMEM((Q, 128), jnp.int32)] * 6
        + [pltpu.VMEM((24, blk), jnp.float32)],
        compiler_params=pltpu.CompilerParams(
            dimension_semantics=("arbitrary",)),
    )(emb, refT_pad)


NC, NS = 2, 16
NW = NC * NS
QPW = Q // NW
RPW = QPW * KNN


def _sc_gather_kernel(idx_hbm, w_hbm, aux_hbm, out_hbm,
                      idx_v, rows_v, w_v, out_v, sem):
    wid = lax.axis_index("s") * NC + lax.axis_index("c")
    base = wid * RPW
    pltpu.sync_copy(idx_hbm.at[pl.ds(base, RPW)], idx_v)
    pltpu.sync_copy(w_hbm.at[pl.ds(base, RPW)], w_v)
    half = RPW // 2
    cp1 = pltpu.async_copy(aux_hbm.at[idx_v.at[pl.ds(0, half)]],
                           rows_v.at[pl.ds(0, half)], sem)
    cp2 = pltpu.async_copy(aux_hbm.at[idx_v.at[pl.ds(half, half)]],
                           rows_v.at[pl.ds(half, half)], sem)
    cp1.wait()
    cp2.wait()
    for q in range(QPW):
        for dd in range(D_AUX // 16):
            sl = pl.ds(dd * 16, 16)
            acc = rows_v[q * KNN, sl] * w_v[q * KNN, sl]
            for t in range(1, KNN):
                acc = acc + rows_v[q * KNN + t, sl] * w_v[q * KNN + t, sl]
            out_v[q, sl] = acc
    pltpu.sync_copy(out_v, out_hbm.at[pl.ds(wid * QPW, QPW)])


def _run_sc_gather(idx_flat, w_rows, aux):
    mesh = plsc.VectorSubcoreMesh(core_axis_name="c", subcore_axis_name="s")
    f = functools.partial(
        pl.kernel,
        out_type=jax.ShapeDtypeStruct((Q, D_AUX), jnp.float32),
        mesh=mesh,
        scratch_types=[
            pltpu.VMEM((RPW,), jnp.int32),
            pltpu.VMEM((RPW, 128), jnp.float32),
            pltpu.VMEM((RPW, D_AUX), jnp.float32),
            pltpu.VMEM((QPW, D_AUX), jnp.float32),
            pltpu.SemaphoreType.DMA,
        ],
    )(_sc_gather_kernel)
    return f(idx_flat, w_rows, aux)


def kernel(embedding_features, reference_embeddings, auxiliary_features):
    emb = embedding_features.reshape(Q, D)
    ref = reference_embeddings.reshape(-1, D)
    k_total = ref.shape[0]

    blk = 4096
    nblk = (k_total + blk - 1) // blk
    kpad = nblk * blk
    refT = ref.T
    refT_pad = jnp.pad(refT, ((0, 0), (0, kpad - k_total)),
                       constant_values=1e18)

    w8, idx8 = _run_topk(emb, refT_pad, nblk, blk, k_total)

    idx_flat = idx8[:, :KNN].reshape(-1)
    w_flat = w8[:, :KNN].reshape(-1)
    w_rows = jnp.broadcast_to(w_flat[:, None], (Q * KNN, D_AUX))

    aux = auxiliary_features.reshape(-1, D_AUX)
    aux_pad = jnp.pad(aux, ((0, 0), (0, 128 - D_AUX)))
    return _run_sc_gather(idx_flat, w_rows, aux_pad)
